# Initial kernel scaffold; baseline (speedup 1.0000x reference)
import jax
import jax.numpy as jnp
from jax.experimental import pallas as pl


def kernel(player_state, item_ids, W1, b1, W2, b2, emb_table, temperature):
    raise NotImplementedError("write your pallas kernel here")


# speedup vs baseline: 1.3259x; 1.3259x over previous
import functools

import jax
import jax.numpy as jnp
from jax.experimental import pallas as pl
from jax.experimental.pallas import tpu as pltpu
from jax.experimental.pallas import tpu_sc as plsc

_B = 16384
_K = 50
_D = 32
_BK = _B * _K

_WINDOW = 128
_NUM_WINDOWS = _BK // _WINDOW


def _sc_gather(emb_table, flat_ids):
    mesh = plsc.VectorSubcoreMesh(core_axis_name="c", subcore_axis_name="s")

    @functools.partial(
        pl.kernel,
        out_type=jax.ShapeDtypeStruct((_BK, _D), jnp.float32),
        mesh=mesh,
        compiler_params=pltpu.CompilerParams(use_tc_tiling_on_sc=False),
    )
    def gather_kernel(tab_hbm, ids_hbm, out_hbm):
        def body(ids_vmem, out_vmem):
            pltpu.sync_copy(tab_hbm.at[ids_vmem.at[0]], out_vmem)

        pltpu.emit_pipeline(
            body,
            grid=(_NUM_WINDOWS,),
            in_specs=[pl.BlockSpec((1, _WINDOW), index_map=lambda i: (0, i))],
            out_specs=[pl.BlockSpec((_WINDOW, _D), index_map=lambda i: (i, 0))],
            core_axis_name=("c", "s"),
            dimension_semantics=(pltpu.PARALLEL,),
        )(ids_hbm, out_hbm)

    return gather_kernel(emb_table, flat_ids)


_BB = 128


def _tc_score(player_state, item_embed, W1, b1, W2, b2, temperature):
    def body(ps_ref, it_ref, w1_ref, b1_ref, w2_ref, b2_ref, t_ref, o_ref):
        h = jnp.maximum(
            jnp.dot(ps_ref[...], w1_ref[...].T,
                    preferred_element_type=jnp.float32) + b1_ref[...],
            0.0,
        )
        pe = jnp.dot(h, w2_ref[...].T,
                     preferred_element_type=jnp.float32) + b2_ref[...]
        it = it_ref[...].reshape(_BB, _K, _D)
        s = jnp.sum(it * pe[:, None, :], axis=-1)
        o_ref[...] = s / t_ref[0]

    return pl.pallas_call(
        body,
        grid=(_B // _BB,),
        in_specs=[
            pl.BlockSpec((_BB, 10), lambda i: (i, 0)),
            pl.BlockSpec((_BB * _K, _D), lambda i: (i, 0)),
            pl.BlockSpec((32, 10), lambda i: (0, 0)),
            pl.BlockSpec((1, 32), lambda i: (0, 0)),
            pl.BlockSpec((_D, 32), lambda i: (0, 0)),
            pl.BlockSpec((1, _D), lambda i: (0, 0)),
            pl.BlockSpec(memory_space=pltpu.SMEM),
        ],
        out_specs=pl.BlockSpec((_BB, _K), lambda i: (i, 0)),
        out_shape=jax.ShapeDtypeStruct((_B, _K), jnp.float32),
    )(player_state, item_embed, W1, b1.reshape(1, 32), W2,
      b2.reshape(1, _D), temperature.reshape(1))


def kernel(player_state, item_ids, W1, b1, W2, b2, emb_table, temperature):
    flat_ids = item_ids.reshape(1, _BK).astype(jnp.int32)
    item_embed = _sc_gather(emb_table, flat_ids)
    return _tc_score(player_state, item_embed, W1, b1, W2, b2, temperature)
